# baseline (speedup 1.0000x reference)
import functools

import jax
import jax.numpy as jnp
from jax import lax
from jax.experimental import pallas as pl
from jax.experimental.pallas import tpu as pltpu
from jax.experimental.pallas import tpu_sc as plsc

VOCAB = 100000
EMBED = 64
B = 4096
T = 200
BFLAT = B * T


@functools.cache
def _build(num_cores: int, num_subcores: int):
    nw = num_cores * num_subcores
    nrows_w = B // nw
    c0 = 128
    c1 = T - c0

    mesh = plsc.VectorSubcoreMesh(core_axis_name="c", subcore_axis_name="s")

    @functools.partial(
        pl.kernel,
        out_type=jax.ShapeDtypeStruct((B, T, EMBED), jnp.float32),
        mesh=mesh,
        scratch_types=[
            pltpu.VMEM((nrows_w, T), jnp.int32),
            pltpu.VMEM((T, EMBED), jnp.float32),
            pltpu.VMEM((T, EMBED), jnp.float32),
            pltpu.SemaphoreType.DMA,
            pltpu.SemaphoreType.DMA,
        ],
        compiler_params=pltpu.CompilerParams(use_tc_tiling_on_sc=False),
    )
    def gather_kernel(tok_hbm, table_hbm, out_hbm, idx_v, buf0, buf1, sem0, sem1):
        wid = lax.axis_index("s") * num_cores + lax.axis_index("c")
        row0 = wid * nrows_w
        pltpu.sync_copy(tok_hbm.at[pl.ds(row0, nrows_w)], idx_v)

        def fire(r, buf, sem):
            pltpu.async_copy(
                table_hbm.at[idx_v.at[r, pl.ds(0, c0)]], buf.at[pl.ds(0, c0)], sem)
            pltpu.async_copy(
                table_hbm.at[idx_v.at[r, pl.ds(c0, c1)]], buf.at[pl.ds(c0, c1)], sem)

        def drain(buf, sem):
            pltpu.make_async_copy(
                table_hbm.at[idx_v.at[0, pl.ds(0, c0)]], buf.at[pl.ds(0, c0)], sem).wait()
            pltpu.make_async_copy(
                table_hbm.at[idx_v.at[0, pl.ds(c0, c1)]], buf.at[pl.ds(c0, c1)], sem).wait()

        fire(0, buf0, sem0)

        @pl.loop(0, nrows_w, step=2)
        def _(r):
            fire(r + 1, buf1, sem1)
            drain(buf0, sem0)
            pltpu.sync_copy(buf0, out_hbm.at[row0 + r])

            @pl.when(r + 2 < nrows_w)
            def _():
                fire(r + 2, buf0, sem0)

            drain(buf1, sem1)
            pltpu.sync_copy(buf1, out_hbm.at[row0 + r + 1])

    return gather_kernel


def kernel(token_ids, table):
    info = plsc.get_sparse_core_info()
    fn = _build(info.num_cores, info.num_subcores)
    return fn(token_ids.astype(jnp.int32), table)
